# Initial kernel scaffold; baseline (speedup 1.0000x reference)
import functools

import jax
import jax.numpy as jnp
from jax import lax
from jax.experimental import pallas as pl
from jax.experimental.pallas import tpu as pltpu
from jax.experimental.pallas import tpu_sc as plsc

DIM = 128
LANES = 16
NVEC = DIM // LANES
NW = 32
EPS = 1e-12


def _rsqrt16(v):
    i = plsc.bitcast(v, jnp.int32)
    i = jnp.int32(0x5F3759DF) - (i >> 1)
    y = plsc.bitcast(i, jnp.float32)
    half = v * 0.5
    for _ in range(3):
        y = y * (1.5 - half * y * y)
    return y


def _body(ids_hbm, tok_hbm, pos_hbm, tte_hbm, gb_hbm, out_hbm,
          idx_v, rows_v, pos_v, tte_v, gb_v, sem):
    chunk = rows_v.shape[0]
    seq = pos_hbm.shape[0]
    wid = lax.axis_index("s") * 2 + lax.axis_index("c")
    base = wid * chunk

    pltpu.sync_copy(ids_hbm.at[pl.ds(base, chunk)], idx_v)
    nstreams = chunk // 128
    cps = []
    for k in range(nstreams):
        cps.append(pltpu.async_copy(
            tok_hbm.at[idx_v.at[k]], rows_v.at[pl.ds(k * 128, 128)], sem))
    pos_base = lax.rem(base, seq)
    pltpu.sync_copy(pos_hbm.at[pl.ds(pos_base, chunk)], pos_v)
    pltpu.sync_copy(tte_hbm.at[pl.ds(0, 1)], tte_v)
    pltpu.sync_copy(gb_hbm, gb_v)
    for cp in cps:
        cp.wait()

    def row(r, carry):
        acc = []
        tot = None
        for j in range(NVEC):
            a = (rows_v[r, pl.ds(LANES * j, LANES)]
                 + pos_v[r, pl.ds(LANES * j, LANES)]
                 + tte_v[0, pl.ds(LANES * j, LANES)])
            acc.append(a)
            tot = a if tot is None else tot + a
        sq = None
        for j in range(NVEC):
            s2 = acc[j] * acc[j]
            sq = s2 if sq is None else sq + s2
        mean = jnp.sum(tot) * (1.0 / DIM)
        msq = jnp.sum(sq) * (1.0 / DIM)
        var = msq - mean * mean
        rstd = _rsqrt16(jnp.full((LANES,), var + EPS, jnp.float32))
        mean_v = jnp.full((LANES,), mean, jnp.float32)
        for j in range(NVEC):
            g = gb_v[0, pl.ds(LANES * j, LANES)]
            b = gb_v[1, pl.ds(LANES * j, LANES)]
            rows_v[r, pl.ds(LANES * j, LANES)] = (
                (acc[j] - mean_v) * rstd * g + b)
        return carry

    lax.fori_loop(0, chunk, row, 0)
    pltpu.sync_copy(rows_v, out_hbm.at[pl.ds(base, chunk)])


@jax.jit
def _run(ids, tok, pos, tte, gb):
    n = ids.shape[0]
    chunk = n // NW
    mesh = plsc.VectorSubcoreMesh(core_axis_name="c", subcore_axis_name="s")
    kern = pl.kernel(
        _body,
        mesh=mesh,
        out_type=jax.ShapeDtypeStruct((n, DIM), jnp.float32),
        scratch_types=[
            pltpu.VMEM((chunk // 128, 128), jnp.int32),
            pltpu.VMEM((chunk, DIM), jnp.float32),
            pltpu.VMEM((chunk, DIM), jnp.float32),
            pltpu.VMEM((1, DIM), jnp.float32),
            pltpu.VMEM((2, DIM), jnp.float32),
            pltpu.SemaphoreType.DMA,
        ],
    )
    return kern(ids, tok, pos, tte, gb)


def kernel(input_ids, token_embedding, position_embeddings,
           token_type_embeddings, ln_gamma, ln_beta):
    b, s = input_ids.shape
    ids = input_ids.reshape(-1).astype(jnp.int32)
    gb = jnp.stack([ln_gamma, ln_beta])
    out = _run(ids, token_embedding, position_embeddings,
               token_type_embeddings, gb)
    return out.reshape(b, s, DIM)


# speedup vs baseline: 7.1930x; 7.1930x over previous
import functools

import jax
import jax.numpy as jnp
from jax import lax
from jax.experimental import pallas as pl
from jax.experimental.pallas import tpu as pltpu
from jax.experimental.pallas import tpu_sc as plsc

DIM = 128
LANES = 16
NVEC = DIM // LANES
NW = 32
EPS = 1e-12


def _rsqrt16(v):
    i = plsc.bitcast(v, jnp.int32)
    i = jnp.int32(0x5F3759DF) - (i >> 1)
    y = plsc.bitcast(i, jnp.float32)
    half = v * 0.5
    for _ in range(3):
        y = y * (1.5 - half * y * y)
    return y


def _body(ids_hbm, tok_hbm, pos_hbm, tte_hbm, gb_hbm, out_hbm,
          idx_v, rows_v, pos_v, tte_v, gb_v, sem):
    chunk = rows_v.shape[0]
    nstreams = chunk // 128
    seq = pos_hbm.shape[0]
    wid = lax.axis_index("s") * 2 + lax.axis_index("c")
    base = wid * chunk

    pltpu.sync_copy(ids_hbm.at[pl.ds(wid * nstreams, nstreams)], idx_v)
    cps = []
    for k in range(nstreams):
        cps.append(pltpu.async_copy(
            tok_hbm.at[idx_v.at[k]], rows_v.at[pl.ds(k * 128, 128)], sem))
    pos_base = lax.rem(base, seq)
    pltpu.sync_copy(pos_hbm.at[pl.ds(pos_base, chunk)], pos_v)
    pltpu.sync_copy(tte_hbm.at[pl.ds(0, 1)], tte_v)
    pltpu.sync_copy(gb_hbm, gb_v)
    for cp in cps:
        cp.wait()

    def row(r, carry):
        acc = []
        tot = None
        for j in range(NVEC):
            a = (rows_v[r, pl.ds(LANES * j, LANES)]
                 + pos_v[r, pl.ds(LANES * j, LANES)]
                 + tte_v[0, pl.ds(LANES * j, LANES)])
            acc.append(a)
            tot = a if tot is None else tot + a
        sq = None
        for j in range(NVEC):
            s2 = acc[j] * acc[j]
            sq = s2 if sq is None else sq + s2
        mean = jnp.sum(tot) * (1.0 / DIM)
        msq = jnp.sum(sq) * (1.0 / DIM)
        var = msq - mean * mean
        rstd = _rsqrt16(jnp.full((LANES,), var + EPS, jnp.float32))
        mean_v = jnp.full((LANES,), mean, jnp.float32)
        for j in range(NVEC):
            g = gb_v[0, pl.ds(LANES * j, LANES)]
            b = gb_v[1, pl.ds(LANES * j, LANES)]
            rows_v[r, pl.ds(LANES * j, LANES)] = (
                (acc[j] - mean_v) * rstd * g + b)
        return carry

    lax.fori_loop(0, chunk, row, 0)
    pltpu.sync_copy(rows_v, out_hbm.at[pl.ds(base, chunk)])


@jax.jit
def _run(ids, tok, pos, tte, gb):
    n = ids.shape[0]
    chunk = n // NW
    ids = ids.reshape(n // 128, 128)
    mesh = plsc.VectorSubcoreMesh(core_axis_name="c", subcore_axis_name="s")
    kern = pl.kernel(
        _body,
        mesh=mesh,
        out_type=jax.ShapeDtypeStruct((n, DIM), jnp.float32),
        scratch_types=[
            pltpu.VMEM((chunk // 128, 128), jnp.int32),
            pltpu.VMEM((chunk, DIM), jnp.float32),
            pltpu.VMEM((chunk, DIM), jnp.float32),
            pltpu.VMEM((1, DIM), jnp.float32),
            pltpu.VMEM((2, DIM), jnp.float32),
            pltpu.SemaphoreType.DMA,
        ],
        compiler_params=pltpu.CompilerParams(needs_layout_passes=False),
    )
    return kern(ids, tok, pos, tte, gb)


def kernel(input_ids, token_embedding, position_embeddings,
           token_type_embeddings, ln_gamma, ln_beta):
    b, s = input_ids.shape
    ids = input_ids.reshape(-1).astype(jnp.int32)
    gb = jnp.stack([ln_gamma, ln_beta])
    out = _run(ids, token_embedding, position_embeddings,
               token_type_embeddings, gb)
    return out.reshape(b, s, DIM)


# speedup vs baseline: 10.9273x; 1.5191x over previous
import jax
import jax.numpy as jnp
import numpy as np
from jax import lax
from jax.experimental import pallas as pl
from jax.experimental.pallas import tpu as pltpu
from jax.experimental.pallas import tpu_sc as plsc

DIM = 128
LANES = 16
NVEC = DIM // LANES
NW = 32
EPS = 1e-12

_GDN = lax.GatherDimensionNumbers(
    offset_dims=(), collapsed_slice_dims=(0,), start_index_map=(0,))


def _shuf(x, idx):
    return lax.gather(x, idx[:, None], _GDN, (1,),
                      mode=lax.GatherScatterMode.PROMISE_IN_BOUNDS)


def _hsum(x, perms):
    for idx in perms:
        x = x + _shuf(x, idx)
    return x


def _rsqrt16(v):
    i = plsc.bitcast(v, jnp.int32)
    i = jnp.int32(0x5F3759DF) - (i >> 1)
    y = plsc.bitcast(i, jnp.float32)
    half = v * 0.5
    for _ in range(2):
        y = y * (1.5 - half * y * y)
    return y


def _body(ids_hbm, tok_hbm, pos_hbm, tte_hbm, gb_hbm, out_hbm,
          idx_v, rows_v, pos_v, tte_v, gb_v, gsem, osem):
    chunk = rows_v.shape[0]
    nstreams = chunk // 128
    seq = pos_hbm.shape[0]
    wid = lax.axis_index("s") * 2 + lax.axis_index("c")
    base = wid * chunk

    pltpu.sync_copy(ids_hbm.at[pl.ds(wid * nstreams, nstreams)], idx_v)
    gcps = [pltpu.async_copy(tok_hbm.at[idx_v.at[k]],
                             rows_v.at[pl.ds(k * 128, 128)], gsem)
            for k in range(nstreams)]
    pos_base = lax.rem(base, seq)
    pltpu.sync_copy(pos_hbm.at[pl.ds(pos_base, chunk)], pos_v)
    pltpu.sync_copy(tte_hbm.at[pl.ds(0, 1)], tte_v)
    pltpu.sync_copy(gb_hbm, gb_v)

    iota = lax.iota(jnp.int32, LANES)
    perms0 = tuple(iota ^ k for k in (8, 4, 2, 1))
    tte0 = tuple(tte_v[0, pl.ds(LANES * j, LANES)] for j in range(NVEC))
    gam0 = tuple(gb_v[0, pl.ds(LANES * j, LANES)] for j in range(NVEC))
    bet0 = tuple(gb_v[1, pl.ds(LANES * j, LANES)] for j in range(NVEC))

    ocps = []
    for k in range(nstreams):
        gcps[k].wait()

        @plsc.parallel_loop(k * 128, (k + 1) * 128, unroll=2,
                            carry=(perms0, tte0, gam0, bet0))
        def row(r, carry):
            perms, tte, gam, bet = carry
            acc = []
            tot = None
            sq = None
            for j in range(NVEC):
                a = (rows_v[r, pl.ds(LANES * j, LANES)]
                     + pos_v[r, pl.ds(LANES * j, LANES)] + tte[j])
                acc.append(a)
                tot = a if tot is None else tot + a
                s2 = a * a
                sq = s2 if sq is None else sq + s2
            tot = _hsum(tot, perms)
            sq = _hsum(sq, perms)
            mean = tot * (1.0 / DIM)
            var = sq * (1.0 / DIM) - mean * mean
            rstd = _rsqrt16(var + EPS)
            for j in range(NVEC):
                rows_v[r, pl.ds(LANES * j, LANES)] = (
                    (acc[j] - mean) * rstd * gam[j] + bet[j])
            return carry

        ocps.append(pltpu.async_copy(
            rows_v.at[pl.ds(k * 128, 128)],
            out_hbm.at[pl.ds(base + k * 128, 128)], osem))
    for cp in ocps:
        cp.wait()


@jax.jit
def _run(ids, tok, pos, tte, gb):
    n = ids.shape[0]
    chunk = n // NW
    ids = ids.reshape(n // 128, 128)
    mesh = plsc.VectorSubcoreMesh(core_axis_name="c", subcore_axis_name="s")
    kern = pl.kernel(
        _body,
        mesh=mesh,
        out_type=jax.ShapeDtypeStruct((n, DIM), jnp.float32),
        scratch_types=[
            pltpu.VMEM((chunk // 128, 128), jnp.int32),
            pltpu.VMEM((chunk, DIM), jnp.float32),
            pltpu.VMEM((chunk, DIM), jnp.float32),
            pltpu.VMEM((1, DIM), jnp.float32),
            pltpu.VMEM((2, DIM), jnp.float32),
            pltpu.SemaphoreType.DMA,
            pltpu.SemaphoreType.DMA,
        ],
        compiler_params=pltpu.CompilerParams(needs_layout_passes=False),
    )
    return kern(ids, tok, pos, tte, gb)


def kernel(input_ids, token_embedding, position_embeddings,
           token_type_embeddings, ln_gamma, ln_beta):
    b, s = input_ids.shape
    ids = input_ids.reshape(-1).astype(jnp.int32)
    gb = jnp.stack([ln_gamma, ln_beta])
    out = _run(ids, token_embedding, position_embeddings,
               token_type_embeddings, gb)
    return out.reshape(b, s, DIM)
